# Initial kernel scaffold; baseline (speedup 1.0000x reference)
import jax
import jax.numpy as jnp
from jax.experimental import pallas as pl


def kernel(dw1, pw1, s1, b1, dw2, pw2, s2, b2, w1x1, x):
    raise NotImplementedError("write your pallas kernel here")


# speedup vs baseline: 8.0682x; 8.0682x over previous
import functools

import jax
import jax.numpy as jnp
from jax.experimental import pallas as pl
from jax.experimental.pallas import tpu as pltpu

_PAD = 256


def _block_kernel(p00, p01, p10, p11, dw1t, w1f, b1, dw2t, w2f, b2, wr,
                  o_ref, y1pad, *, ow, m, ch):
    lw = 2 * ow
    nch = m // ch
    lane = jax.lax.broadcasted_iota(jnp.int32, (1, ch), 1)
    col = jnp.bitwise_and(lane, ow - 1)
    mask_l = jnp.where(col == 0, 0.0, 1.0)
    mask_r = jnp.where(col == ow - 1, 0.0, 1.0)

    def shifted(ref, dh, dw, c0, rank3=False):
        s = _PAD + lw * dh + dw + c0
        t = ref[0, :, s:s + ch] if rank3 else ref[:, s:s + ch]
        if dw == -1:
            t = t * mask_l
        elif dw == 1:
            t = t * mask_r
        return t

    phase = {(0, 0): p00, (0, 1): p01, (1, 0): p10, (1, 1): p11}

    def tap1(k):
        return (0, 0) if k == 1 else ((1, -1) if k == 0 else (1, 0))

    y1pad[:, :_PAD] = jnp.zeros_like(y1pad[:, :_PAD])
    y1pad[:, _PAD + m:] = jnp.zeros_like(y1pad[:, _PAD + m:])
    for ci in range(nch):
        c0 = ci * ch
        d1 = None
        for kh in range(3):
            a, dh = tap1(kh)
            for kw in range(3):
                b, dw = tap1(kw)
                c = dw1t[:, kh * 3 + kw][:, None] * shifted(
                    phase[(a, b)], dh, dw, c0, rank3=True)
                d1 = c if d1 is None else d1 + c
        y1 = jnp.dot(w1f[...], d1, preferred_element_type=jnp.float32) + b1[...]
        y1pad[:, _PAD + c0:_PAD + c0 + ch] = jnp.maximum(y1, 0.0)

    for ci in range(nch):
        c0 = ci * ch
        d2 = None
        for kh in range(3):
            for kw in range(3):
                c = dw2t[:, kh * 3 + kw][:, None] * shifted(
                    y1pad, kh - 1, kw - 1, c0)
                d2 = c if d2 is None else d2 + c
        y2 = jnp.dot(w2f[...], d2, preferred_element_type=jnp.float32) + b2[...]
        res = jnp.dot(wr[...], p00[0, :, _PAD + c0:_PAD + c0 + ch],
                      preferred_element_type=jnp.float32)
        o_ref[0, :, c0:c0 + ch] = jnp.maximum(y2 + res, 0.0)


def kernel(dw1, pw1, s1, b1, dw2, pw2, s2, b2, w1x1, x):
    n, cin, h, w = x.shape
    oh, ow = h // 2, w // 2
    p = n // 2
    m = oh * 2 * ow
    cout = pw1.shape[1]

    xr = x.reshape(p, 2, cin, oh, 2, ow, 2)

    def mkphase(a, b):
        ph = xr[:, :, :, :, a, :, b]
        ph = ph.transpose(0, 2, 3, 1, 4).reshape(p, cin, m)
        return jnp.pad(ph, ((0, 0), (0, 0), (_PAD, 0)))

    phases = [mkphase(0, 0), mkphase(0, 1), mkphase(1, 0), mkphase(1, 1)]

    w1f = pw1.T * s1[:, None]
    w2f = pw2.T * s2[:, None]
    wr = w1x1.T
    b1r = b1[:, None]
    b2r = b2[:, None]
    dw1t = dw1.T
    dw2t = dw2.T

    ch = min(m, 2048)
    body = functools.partial(_block_kernel, ow=ow, m=m, ch=ch)

    stream = lambda c: pl.BlockSpec((1, c, _PAD + m), lambda i: (i, 0, 0))
    resident = lambda s: pl.BlockSpec(s, lambda i: (0, 0))

    out = pl.pallas_call(
        body,
        out_shape=jax.ShapeDtypeStruct((p, cout, m), jnp.float32),
        grid=(p,),
        in_specs=[stream(cin)] * 4 + [
            resident(dw1t.shape), resident(w1f.shape), resident(b1r.shape),
            resident(dw2t.shape), resident(w2f.shape), resident(b2r.shape),
            resident(wr.shape),
        ],
        out_specs=pl.BlockSpec((1, cout, m), lambda i: (i, 0, 0)),
        scratch_shapes=[pltpu.VMEM((cout, _PAD + m + 256), jnp.float32)],
        compiler_params=pltpu.CompilerParams(
            dimension_semantics=("parallel",),
            vmem_limit_bytes=50 * 1024 * 1024,
        ),
    )(*phases, dw1t, w1f, b1r, dw2t, w2f, b2r, wr)

    out = out.reshape(p, cout, oh, 2, ow).transpose(0, 3, 1, 2, 4)
    return out.reshape(n, cout, oh, ow)


# speedup vs baseline: 10.5766x; 1.3109x over previous
import functools

import jax
import jax.numpy as jnp
from jax.experimental import pallas as pl
from jax.experimental.pallas import tpu as pltpu

_PAD = 256


def _block_kernel(xall, dw1t, w1f, b1, dw2t, w2f, b2, wr, o_ref, y1pad,
                  *, cin, ow, m, ch):
    lw = 2 * ow
    nch = m // ch
    lane = jax.lax.broadcasted_iota(jnp.int32, (1, ch), 1)
    col = jnp.bitwise_and(lane, ow - 1)
    mask_l = jnp.where(col == 0, 0.0, 1.0)
    mask_r = jnp.where(col == ow - 1, 0.0, 1.0)

    def masked(t, dw):
        if dw == -1:
            return t * mask_l
        if dw == 1:
            return t * mask_r
        return t

    def tap_x(idx, dh, dw, c0):
        s = c0 + lw * dh + dw
        if s >= 0:
            t = xall[idx, 0, :, s:s + ch]
        else:
            t = jnp.concatenate(
                [jnp.zeros((cin, -s), jnp.float32), xall[idx, 0, :, 0:ch + s]],
                axis=-1)
        return masked(t, dw)

    def tap_y(dh, dw, c0):
        s = _PAD + c0 + lw * dh + dw
        return masked(y1pad[:, s:s + ch], dw)

    def tap1(k):
        return (0, 0) if k == 1 else ((1, -1) if k == 0 else (1, 0))

    y1pad[:, :_PAD] = jnp.zeros_like(y1pad[:, :_PAD])
    y1pad[:, _PAD + m:] = jnp.zeros_like(y1pad[:, _PAD + m:])
    for ci in range(nch):
        c0 = ci * ch
        d1 = None
        for kh in range(3):
            a, dh = tap1(kh)
            for kw in range(3):
                b, dw = tap1(kw)
                c = dw1t[:, kh * 3 + kw][:, None] * tap_x(2 * a + b, dh, dw, c0)
                d1 = c if d1 is None else d1 + c
        y1 = jnp.dot(w1f[...], d1, preferred_element_type=jnp.float32) + b1[...]
        y1pad[:, _PAD + c0:_PAD + c0 + ch] = jnp.maximum(y1, 0.0)

    for ci in range(nch):
        c0 = ci * ch
        d2 = None
        for kh in range(3):
            for kw in range(3):
                c = dw2t[:, kh * 3 + kw][:, None] * tap_y(kh - 1, kw - 1, c0)
                d2 = c if d2 is None else d2 + c
        y2 = jnp.dot(w2f[...], d2, preferred_element_type=jnp.float32) + b2[...]
        res = jnp.dot(wr[...], xall[0, 0, :, c0:c0 + ch],
                      preferred_element_type=jnp.float32)
        o_ref[0, :, c0:c0 + ch] = jnp.maximum(y2 + res, 0.0)


def kernel(dw1, pw1, s1, b1, dw2, pw2, s2, b2, w1x1, x):
    n, cin, h, w = x.shape
    oh, ow = h // 2, w // 2
    p = n // 2
    m = oh * 2 * ow
    cout = pw1.shape[1]

    xr = x.reshape(p, 2, cin, oh, 2, ow, 2)
    xall = xr.transpose(4, 6, 0, 2, 3, 1, 5)
    xall = xall.reshape(4, p, cin, m)

    w1f = pw1.T * s1[:, None]
    w2f = pw2.T * s2[:, None]
    wr = w1x1.T
    b1r = b1[:, None]
    b2r = b2[:, None]
    dw1t = dw1.T
    dw2t = dw2.T

    ch = min(m, 4096)
    body = functools.partial(_block_kernel, cin=cin, ow=ow, m=m, ch=ch)

    resident = lambda s: pl.BlockSpec(s, lambda i: (0, 0))

    out = pl.pallas_call(
        body,
        out_shape=jax.ShapeDtypeStruct((p, cout, m), jnp.float32),
        grid=(p,),
        in_specs=[pl.BlockSpec((4, 1, cin, m), lambda i: (0, i, 0, 0))] + [
            resident(dw1t.shape), resident(w1f.shape), resident(b1r.shape),
            resident(dw2t.shape), resident(w2f.shape), resident(b2r.shape),
            resident(wr.shape),
        ],
        out_specs=pl.BlockSpec((1, cout, m), lambda i: (i, 0, 0)),
        scratch_shapes=[pltpu.VMEM((cout, _PAD + m + 256), jnp.float32)],
        compiler_params=pltpu.CompilerParams(
            dimension_semantics=("parallel",),
            vmem_limit_bytes=50 * 1024 * 1024,
        ),
    )(xall, dw1t, w1f, b1r, dw2t, w2f, b2r, wr)

    out = out.reshape(p, cout, oh, 2, ow).transpose(0, 3, 1, 2, 4)
    return out.reshape(n, cout, oh, ow)


# speedup vs baseline: 16.3229x; 1.5433x over previous
import functools

import jax
import jax.numpy as jnp
from jax.experimental import pallas as pl
from jax.experimental.pallas import tpu as pltpu

_PAD = 256


def _block_kernel(xall, dw1t, w1f, b1, dw2t, w2f, b2, wr, o_ref, y1pad,
                  *, cin, oh, ow, m, ch):
    hw = oh * ow
    nch = m // ch
    lane = jax.lax.broadcasted_iota(jnp.int32, (1, ch), 1)
    col = jnp.bitwise_and(lane, ow - 1)
    row = jnp.bitwise_and(jax.lax.shift_right_logical(lane, ow.bit_length() - 1),
                          oh - 1)
    colm = {-1: col != 0, 0: None, 1: col != ow - 1}
    rowm = {-1: row != 0, 0: None, 1: row != oh - 1}
    masks = {}
    for dh in (-1, 0, 1):
        for dw in (-1, 0, 1):
            sel = rowm[dh] if dw == 0 else (
                colm[dw] if dh == 0 else rowm[dh] & colm[dw])
            masks[(dh, dw)] = (None if sel is None
                               else jnp.where(sel, 1.0, 0.0))

    def masked(t, dh, dw):
        mk = masks[(dh, dw)]
        return t if mk is None else t * mk

    def tap_x(idx, dh, dw, c0):
        s = c0 + ow * dh + dw
        if s >= 0:
            t = xall[idx, 0, :, s:s + ch]
        else:
            t = jnp.concatenate(
                [jnp.zeros((cin, -s), jnp.float32), xall[idx, 0, :, 0:ch + s]],
                axis=-1)
        return masked(t, dh, dw)

    def tap_y(dh, dw, c0):
        s = _PAD + c0 + ow * dh + dw
        return masked(y1pad[:, s:s + ch], dh, dw)

    def tap1(k):
        return (0, 0) if k == 1 else ((1, -1) if k == 0 else (1, 0))

    y1pad[:, :_PAD] = jnp.zeros_like(y1pad[:, :_PAD])
    y1pad[:, _PAD + m:] = jnp.zeros_like(y1pad[:, _PAD + m:])
    for ci in range(nch):
        c0 = ci * ch
        d1 = None
        for kh in range(3):
            a, dh = tap1(kh)
            for kw in range(3):
                b, dw = tap1(kw)
                c = dw1t[:, kh * 3 + kw][:, None] * tap_x(2 * a + b, dh, dw, c0)
                d1 = c if d1 is None else d1 + c
        y1 = jnp.dot(w1f[...], d1, preferred_element_type=jnp.float32) + b1[...]
        y1pad[:, _PAD + c0:_PAD + c0 + ch] = jnp.maximum(y1, 0.0)

    for ci in range(nch):
        c0 = ci * ch
        d2 = None
        for kh in range(3):
            for kw in range(3):
                c = dw2t[:, kh * 3 + kw][:, None] * tap_y(kh - 1, kw - 1, c0)
                d2 = c if d2 is None else d2 + c
        y2 = jnp.dot(w2f[...], d2, preferred_element_type=jnp.float32) + b2[...]
        res = jnp.dot(wr[...], xall[0, 0, :, c0:c0 + ch],
                      preferred_element_type=jnp.float32)
        q, r0 = divmod(c0, hw)
        o_ref[0, q, :, r0:r0 + ch] = jnp.maximum(y2 + res, 0.0)


def kernel(dw1, pw1, s1, b1, dw2, pw2, s2, b2, w1x1, x):
    n, cin, h, w = x.shape
    oh, ow = h // 2, w // 2
    hw = oh * ow
    p = n // 2
    m = 2 * hw
    cout = pw1.shape[1]

    xr = x.reshape(p, 2, cin, oh, 2, ow, 2)
    xall = xr.transpose(4, 6, 0, 2, 1, 3, 5)
    xall = xall.reshape(4, p, cin, m)

    w1f = pw1.T * s1[:, None]
    w2f = pw2.T * s2[:, None]
    wr = w1x1.T
    b1r = b1[:, None]
    b2r = b2[:, None]
    dw1t = dw1.T
    dw2t = dw2.T

    ch = min(hw, 4096)
    body = functools.partial(_block_kernel, cin=cin, oh=oh, ow=ow, m=m, ch=ch)

    resident = lambda s: pl.BlockSpec(s, lambda i: (0, 0))

    out = pl.pallas_call(
        body,
        out_shape=jax.ShapeDtypeStruct((p, 2, cout, hw), jnp.float32),
        grid=(p,),
        in_specs=[pl.BlockSpec((4, 1, cin, m), lambda i: (0, i, 0, 0))] + [
            resident(dw1t.shape), resident(w1f.shape), resident(b1r.shape),
            resident(dw2t.shape), resident(w2f.shape), resident(b2r.shape),
            resident(wr.shape),
        ],
        out_specs=pl.BlockSpec((1, 2, cout, hw), lambda i: (i, 0, 0, 0)),
        scratch_shapes=[pltpu.VMEM((cout, _PAD + m + 256), jnp.float32)],
        compiler_params=pltpu.CompilerParams(
            dimension_semantics=("parallel",),
            vmem_limit_bytes=50 * 1024 * 1024,
        ),
    )(xall, dw1t, w1f, b1r, dw2t, w2f, b2r, wr)

    return out.reshape(n, cout, oh, ow)
